# Initial kernel scaffold; baseline (speedup 1.0000x reference)
import functools

import jax
import jax.numpy as jnp
from jax import lax
from jax.experimental import pallas as pl
from jax.experimental.pallas import tpu as pltpu
from jax.experimental.pallas import tpu_sc as plsc

B = 50000
D = 128
S = 10
L = 16
NW = 32
BLK = 112
NBLK = 14
CHUNK = BLK * NBLK
B_PAD = NW * CHUNK


@functools.partial(
    pl.kernel,
    out_type=[
        jax.ShapeDtypeStruct((B_PAD, D), jnp.float32),
        jax.ShapeDtypeStruct((B_PAD, D), jnp.float32),
    ],
    mesh=plsc.VectorSubcoreMesh(core_axis_name="c", subcore_axis_name="s"),
    scratch_types=[
        pltpu.VMEM((BLK,), jnp.int32),
        pltpu.VMEM((BLK, D), jnp.float32),
        pltpu.VMEM((BLK, D), jnp.float32),
        pltpu.SemaphoreType.DMA,
    ],
)
def _sc_gather(nodes_hbm, neigh_hbm, feat_hbm, self_out, sum_out,
               idx_v, rows_v, acc_v, sem):
    wid = lax.axis_index("s") * 2 + lax.axis_index("c")
    wbase = wid * CHUNK

    def block(b, carry):
        base = wbase + b * BLK
        pltpu.sync_copy(nodes_hbm.at[pl.ds(base, BLK)], idx_v)
        pltpu.async_copy(feat_hbm.at[idx_v], rows_v, sem).wait()
        pltpu.sync_copy(rows_v, self_out.at[pl.ds(base, BLK)])
        pltpu.sync_copy(neigh_hbm.at[0, pl.ds(base, BLK)], idx_v)
        pltpu.async_copy(feat_hbm.at[idx_v], acc_v, sem).wait()
        for j in range(1, S):
            pltpu.sync_copy(neigh_hbm.at[j, pl.ds(base, BLK)], idx_v)
            pltpu.async_copy(feat_hbm.at[idx_v], rows_v, sem).wait()

            def row(r, c2):
                for c in range(D // L):
                    sl = pl.ds(c * L, L)
                    acc_v[r, sl] += rows_v[r, sl]
                return c2

            lax.fori_loop(0, BLK, row, 0)
        pltpu.sync_copy(acc_v, sum_out.at[pl.ds(base, BLK)])
        return carry

    lax.fori_loop(0, NBLK, block, 0)


MB = 512


def _mm_body(self_ref, sum_ref, w_ref, o_ref):
    w1 = w_ref[:D, :]
    w2 = w_ref[D:, :]
    x1 = self_ref[...]
    x2 = sum_ref[...] * jnp.float32(1.0 / S)
    acc = jnp.dot(x1, w1, preferred_element_type=jnp.float32)
    acc += jnp.dot(x2, w2, preferred_element_type=jnp.float32)
    o_ref[...] = jnp.maximum(acc, 0.0)


def kernel(nodes, neigh_idx, features, weight):
    pad = B_PAD - B
    nodes_p = jnp.pad(nodes, (0, pad))
    neigh_t = jnp.pad(neigh_idx, ((0, pad), (0, 0))).T
    self_feats, neigh_sum = _sc_gather(nodes_p, neigh_t, features)
    out = pl.pallas_call(
        _mm_body,
        grid=(B_PAD // MB,),
        in_specs=[
            pl.BlockSpec((MB, D), lambda i: (i, 0)),
            pl.BlockSpec((MB, D), lambda i: (i, 0)),
            pl.BlockSpec((2 * D, D), lambda i: (0, 0)),
        ],
        out_specs=pl.BlockSpec((MB, D), lambda i: (i, 0)),
        out_shape=jax.ShapeDtypeStruct((B, D), jnp.float32),
    )(self_feats, neigh_sum, weight)
    return out


# speedup vs baseline: 3.4583x; 3.4583x over previous
import functools

import jax
import jax.numpy as jnp
from jax import lax
from jax.experimental import pallas as pl
from jax.experimental.pallas import tpu as pltpu
from jax.experimental.pallas import tpu_sc as plsc

B = 50000
D = 128
S = 10
L = 16
NW = 32
BLK = 112
NBLK = 14
CHUNK = BLK * NBLK
B_PAD = NW * CHUNK


@functools.cache
def _make_sc_gather():
    @functools.partial(
        pl.kernel,
        out_type=[
            jax.ShapeDtypeStruct((B_PAD, D), jnp.float32),
            jax.ShapeDtypeStruct((B_PAD, D), jnp.float32),
        ],
        mesh=plsc.VectorSubcoreMesh(core_axis_name="c", subcore_axis_name="s"),
        scratch_types=[
            pltpu.VMEM((BLK,), jnp.int32),
            pltpu.VMEM((BLK, D), jnp.float32),
            pltpu.VMEM((BLK, D), jnp.float32),
            pltpu.SemaphoreType.DMA,
        ],
    )
    def _sc_gather(nodes_hbm, neigh_hbm, feat_hbm, self_out, sum_out,
                   idx_v, rows_v, acc_v, sem):
        _sc_gather_body(nodes_hbm, neigh_hbm, feat_hbm, self_out, sum_out,
                        idx_v, rows_v, acc_v, sem)

    return _sc_gather


def _sc_gather_body(nodes_hbm, neigh_hbm, feat_hbm, self_out, sum_out,
                    idx_v, rows_v, acc_v, sem):
    wid = lax.axis_index("s") * 2 + lax.axis_index("c")
    wbase = wid * CHUNK

    def block(b, carry):
        base = wbase + b * BLK
        pltpu.sync_copy(nodes_hbm.at[pl.ds(base, BLK)], idx_v)
        pltpu.async_copy(feat_hbm.at[idx_v], rows_v, sem).wait()
        pltpu.sync_copy(rows_v, self_out.at[pl.ds(base, BLK)])
        pltpu.sync_copy(neigh_hbm.at[pl.ds(base, BLK)], idx_v)
        pltpu.async_copy(feat_hbm.at[idx_v], acc_v, sem).wait()
        for j in range(1, S):
            pltpu.sync_copy(neigh_hbm.at[pl.ds(j * B_PAD + base, BLK)], idx_v)
            pltpu.async_copy(feat_hbm.at[idx_v], rows_v, sem).wait()

            def row(r, c2):
                for c in range(D // L):
                    sl = pl.ds(c * L, L)
                    acc_v[r, sl] += rows_v[r, sl]
                return c2

            lax.fori_loop(0, BLK, row, 0)
        pltpu.sync_copy(acc_v, sum_out.at[pl.ds(base, BLK)])
        return carry

    lax.fori_loop(0, NBLK, block, 0)


MB = 512


def _mm_body(self_ref, sum_ref, w_ref, o_ref):
    w1 = w_ref[:D, :]
    w2 = w_ref[D:, :]
    x1 = self_ref[...]
    x2 = sum_ref[...] * jnp.float32(1.0 / S)
    acc = jnp.dot(x1, w1, preferred_element_type=jnp.float32)
    acc += jnp.dot(x2, w2, preferred_element_type=jnp.float32)
    o_ref[...] = jnp.maximum(acc, 0.0)


def kernel(nodes, neigh_idx, features, weight):
    pad = B_PAD - B
    nodes_p = jnp.pad(nodes, (0, pad))
    neigh_t = jnp.pad(neigh_idx, ((0, pad), (0, 0))).T.reshape(-1)
    self_feats, neigh_sum = _make_sc_gather()(nodes_p, neigh_t, features)
    out = pl.pallas_call(
        _mm_body,
        grid=(B_PAD // MB,),
        in_specs=[
            pl.BlockSpec((MB, D), lambda i: (i, 0)),
            pl.BlockSpec((MB, D), lambda i: (i, 0)),
            pl.BlockSpec((2 * D, D), lambda i: (0, 0)),
        ],
        out_specs=pl.BlockSpec((MB, D), lambda i: (i, 0)),
        out_shape=jax.ShapeDtypeStruct((B, D), jnp.float32),
    )(self_feats, neigh_sum, weight)
    return out


# speedup vs baseline: 4.3713x; 1.2640x over previous
import functools

import jax
import jax.numpy as jnp
from jax import lax
from jax.experimental import pallas as pl
from jax.experimental.pallas import tpu as pltpu
from jax.experimental.pallas import tpu_sc as plsc

B = 50000
D = 128
S = 10
L = 16
NW = 32
BLK = 112
NBLK = 14
CHUNK = BLK * NBLK
B_PAD = NW * CHUNK


@functools.cache
def _make_sc_gather():
    @functools.partial(
        pl.kernel,
        out_type=[
            jax.ShapeDtypeStruct((B_PAD, D), jnp.float32),
            jax.ShapeDtypeStruct((B_PAD, D), jnp.float32),
        ],
        mesh=plsc.VectorSubcoreMesh(core_axis_name="c", subcore_axis_name="s"),
        scratch_types=[
            pltpu.VMEM((BLK,), jnp.int32),
            pltpu.VMEM((BLK, D), jnp.float32),
            pltpu.VMEM((BLK, D), jnp.float32),
            pltpu.SemaphoreType.DMA,
        ],
    )
    def _sc_gather(nodes_hbm, neigh_hbm, feat_hbm, self_out, sum_out,
                   idx_v, rows_v, acc_v, sem):
        _sc_gather_body(nodes_hbm, neigh_hbm, feat_hbm, self_out, sum_out,
                        idx_v, rows_v, acc_v, sem)

    return _sc_gather


def _sc_gather_body(nodes_hbm, neigh_hbm, feat_hbm, self_out, sum_out,
                    idx_v, rows_v, acc_v, sem):
    wid = lax.axis_index("s") * 2 + lax.axis_index("c")
    wbase = wid * CHUNK

    def block(b, carry):
        base = wbase + b * BLK
        pltpu.sync_copy(nodes_hbm.at[pl.ds(base, BLK)], idx_v)
        pltpu.async_copy(feat_hbm.at[idx_v], rows_v, sem).wait()
        pltpu.sync_copy(rows_v, self_out.at[pl.ds(base, BLK)])
        pltpu.sync_copy(neigh_hbm.at[pl.ds(base, BLK)], idx_v)
        pltpu.async_copy(feat_hbm.at[idx_v], acc_v, sem).wait()
        for j in range(1, S):
            pltpu.sync_copy(neigh_hbm.at[pl.ds(j * B_PAD + base, BLK)], idx_v)
            pltpu.async_copy(feat_hbm.at[idx_v], acc_v, sem, add=True).wait()
        pltpu.sync_copy(acc_v, sum_out.at[pl.ds(base, BLK)])
        return carry

    lax.fori_loop(0, NBLK, block, 0)


MB = 512


def _mm_body(self_ref, sum_ref, w_ref, o_ref):
    w1 = w_ref[:D, :]
    w2 = w_ref[D:, :]
    x1 = self_ref[...]
    x2 = sum_ref[...] * jnp.float32(1.0 / S)
    acc = jnp.dot(x1, w1, preferred_element_type=jnp.float32)
    acc += jnp.dot(x2, w2, preferred_element_type=jnp.float32)
    o_ref[...] = jnp.maximum(acc, 0.0)


def kernel(nodes, neigh_idx, features, weight):
    pad = B_PAD - B
    nodes_p = jnp.pad(nodes, (0, pad))
    neigh_t = jnp.pad(neigh_idx, ((0, pad), (0, 0))).T.reshape(-1)
    self_feats, neigh_sum = _make_sc_gather()(nodes_p, neigh_t, features)
    out = pl.pallas_call(
        _mm_body,
        grid=(B_PAD // MB,),
        in_specs=[
            pl.BlockSpec((MB, D), lambda i: (i, 0)),
            pl.BlockSpec((MB, D), lambda i: (i, 0)),
            pl.BlockSpec((2 * D, D), lambda i: (0, 0)),
        ],
        out_specs=pl.BlockSpec((MB, D), lambda i: (i, 0)),
        out_shape=jax.ShapeDtypeStruct((B, D), jnp.float32),
    )(self_feats, neigh_sum, weight)
    return out


# speedup vs baseline: 6.4704x; 1.4802x over previous
import functools

import jax
import jax.numpy as jnp
from jax import lax
from jax.experimental import pallas as pl
from jax.experimental.pallas import tpu as pltpu
from jax.experimental.pallas import tpu_sc as plsc

B = 50000
D = 128
S = 10
L = 16
NW = 32
BLK = 112
NBLK = 14
CHUNK = BLK * NBLK
B_PAD = NW * CHUNK
NIDX = NBLK * (S + 1)


@functools.cache
def _make_sc_gather():
    @functools.partial(
        pl.kernel,
        out_type=[
            jax.ShapeDtypeStruct((B_PAD, D), jnp.float32),
            jax.ShapeDtypeStruct((B_PAD, D), jnp.float32),
        ],
        mesh=plsc.VectorSubcoreMesh(core_axis_name="c", subcore_axis_name="s"),
        scratch_types=[
            pltpu.VMEM((NIDX, BLK), jnp.int32),
            pltpu.VMEM((2, BLK, D), jnp.float32),
            pltpu.VMEM((2, BLK, D), jnp.float32),
            pltpu.SemaphoreType.DMA,
            pltpu.SemaphoreType.DMA,
            pltpu.SemaphoreType.DMA,
            pltpu.SemaphoreType.DMA,
            pltpu.SemaphoreType.DMA,
            pltpu.SemaphoreType.DMA,
            pltpu.SemaphoreType.DMA,
            pltpu.SemaphoreType.DMA,
        ],
    )
    def _sc_gather(idx_hbm, feat_hbm, self_out, sum_out,
                   idx_v, selfb, accb, sgs0, sgs1, sga0, sga1,
                   sos0, sos1, soa0, soa1):
        sg_self = (sgs0, sgs1)
        sg_acc = (sga0, sga1)
        so_self = (sos0, sos1)
        so_acc = (soa0, soa1)
        wid = lax.axis_index("s") * 2 + lax.axis_index("c")
        wbase = wid * CHUNK

        pltpu.sync_copy(idx_hbm.at[wid], idx_v)

        def zero_acc(p):
            zeros = jnp.zeros((L,), jnp.float32)

            def zrow(r, c2):
                for c in range(D // L):
                    accb[p, r, pl.ds(c * L, L)] = zeros
                return c2

            lax.fori_loop(0, BLK, zrow, 0)

        def fire_block(b):
            p = b & 1
            zero_acc(p)
            ds = pltpu.async_copy(
                feat_hbm.at[idx_v.at[b * (S + 1)]], selfb.at[p], sg_self[p])
            da = [
                pltpu.async_copy(
                    feat_hbm.at[idx_v.at[b * (S + 1) + 1 + j]], accb.at[p],
                    sg_acc[p], add=True)
                for j in range(S)
            ]
            return ds, da

        def retire_block(b, gathers):
            p = b & 1
            base = wbase + b * BLK
            ds, da = gathers
            ds.wait()
            os = pltpu.async_copy(selfb.at[p], self_out.at[pl.ds(base, BLK)],
                                  so_self[p])
            for d in da:
                d.wait()
            oa = pltpu.async_copy(accb.at[p], sum_out.at[pl.ds(base, BLK)],
                                  so_acc[p])
            return os, oa

        outs = [None, None]
        gathers = fire_block(0)
        for b in range(NBLK):
            nxt = None
            if b + 1 < NBLK:
                p = (b + 1) & 1
                if outs[p] is not None:
                    outs[p][0].wait()
                    outs[p][1].wait()
                nxt = fire_block(b + 1)
            outs[b & 1] = retire_block(b, gathers)
            gathers = nxt
        outs[0][0].wait()
        outs[0][1].wait()
        outs[1][0].wait()
        outs[1][1].wait()

    return _sc_gather


MB = 512


def _mm_body(self_ref, sum_ref, w_ref, o_ref):
    w1 = w_ref[:D, :]
    w2 = w_ref[D:, :]
    x1 = self_ref[...]
    x2 = sum_ref[...] * jnp.float32(1.0 / S)
    acc = jnp.dot(x1, w1, preferred_element_type=jnp.float32)
    acc += jnp.dot(x2, w2, preferred_element_type=jnp.float32)
    o_ref[...] = jnp.maximum(acc, 0.0)


def kernel(nodes, neigh_idx, features, weight):
    pad = B_PAD - B
    nodes_r = jnp.pad(nodes, (0, pad)).reshape(NW, NBLK, 1, BLK)
    neigh_r = (jnp.pad(neigh_idx, ((0, pad), (0, 0)))
               .reshape(NW, NBLK, BLK, S)
               .transpose(0, 1, 3, 2))
    idx_all = jnp.concatenate([nodes_r, neigh_r], axis=2).reshape(NW, NIDX, BLK)
    self_feats, neigh_sum = _make_sc_gather()(idx_all, features)
    out = pl.pallas_call(
        _mm_body,
        grid=(B_PAD // MB,),
        in_specs=[
            pl.BlockSpec((MB, D), lambda i: (i, 0)),
            pl.BlockSpec((MB, D), lambda i: (i, 0)),
            pl.BlockSpec((2 * D, D), lambda i: (0, 0)),
        ],
        out_specs=pl.BlockSpec((MB, D), lambda i: (i, 0)),
        out_shape=jax.ShapeDtypeStruct((B, D), jnp.float32),
    )(self_feats, neigh_sum, weight)
    return out
